# baseline (speedup 1.0000x reference)
import jax
import jax.numpy as jnp
import numpy as np
from jax.experimental import pallas as pl
from jax.experimental.pallas import tpu as pltpu

_G = 256


def _reduce_body(f_ref, o_ref):
    f = f_ref[...]
    mean = jnp.mean(f, axis=1, keepdims=True)
    mx = jnp.max(f, axis=1, keepdims=True)
    o_ref[...] = jnp.concatenate([mean, mx], axis=1)


def _apply_body(f_ref, conv_ref, o_ref):
    att = jax.nn.sigmoid(conv_ref[...])
    o_ref[...] = f_ref[...] * att


def _rowblocks(n, b):
    return (n + b - 1) // b


def kernel(features, indices, W):
    n, c = features.shape
    B = 512

    feat2 = pl.pallas_call(
        _reduce_body,
        grid=(_rowblocks(n, B),),
        in_specs=[pl.BlockSpec((B, c), lambda i: (i, 0))],
        out_specs=pl.BlockSpec((B, 2), lambda i: (i, 0)),
        out_shape=jax.ShapeDtypeStruct((n, 2), features.dtype),
    )(features)

    idx = indices.astype(jnp.uint32)
    b_, z, y, x = idx[:, 0], idx[:, 1], idx[:, 2], idx[:, 3]
    keys = ((b_ * _G + z) * _G + y) * _G + x

    order = jnp.argsort(keys)
    skeys = keys[order]
    sfeat2 = feat2[order]
    sa, sm = sfeat2[:, 0], sfeat2[:, 1]

    offs = np.array([(dz, dy, dx)
                     for dz in (-1, 0, 1) for dy in (-1, 0, 1) for dx in (-1, 0, 1)],
                    dtype=np.int64)
    deltas = jnp.asarray(
        (offs[:, 0] * _G * _G + offs[:, 1] * _G + offs[:, 2]).astype(np.uint32))
    dz = jnp.asarray(offs[:, 0].astype(np.int32))
    dy = jnp.asarray(offs[:, 1].astype(np.int32))
    dx = jnp.asarray(offs[:, 2].astype(np.int32))

    zi = z.astype(jnp.int32)[None, :]
    yi = y.astype(jnp.int32)[None, :]
    xi = x.astype(jnp.int32)[None, :]
    nz = zi + dz[:, None]
    ny = yi + dy[:, None]
    nx = xi + dx[:, None]
    valid = ((nz >= 0) & (nz < _G) & (ny >= 0) & (ny < _G)
             & (nx >= 0) & (nx < _G))

    nk = keys[None, :] + deltas[:, None]
    pos = jnp.searchsorted(skeys, nk)
    pos = jnp.clip(pos, 0, n - 1)
    match = (skeys[pos] == nk) & valid

    wa = W[:, 0, 0]
    wm = W[:, 1, 0]
    contrib = (sa[pos] * wa[:, None] + sm[pos] * wm[:, None])
    conv = jnp.sum(jnp.where(match, contrib, 0.0), axis=0)[:, None]

    out = pl.pallas_call(
        _apply_body,
        grid=(_rowblocks(n, B),),
        in_specs=[pl.BlockSpec((B, c), lambda i: (i, 0)),
                  pl.BlockSpec((B, 1), lambda i: (i, 0))],
        out_specs=pl.BlockSpec((B, c), lambda i: (i, 0)),
        out_shape=jax.ShapeDtypeStruct((n, c), features.dtype),
    )(features, conv)
    return out
